# baseline (speedup 1.0000x reference)
import functools

import jax
import jax.numpy as jnp
from jax import lax
from jax.experimental import pallas as pl
from jax.experimental.pallas import tpu as pltpu
from jax.experimental.pallas import tpu_sc as plsc

N = 10000
E = 160000
E_PAD = 163840
V_PAD = 10240
NC, NS = 2, 16
NW = NC * NS
CHUNK = 128


def _mesh():
    return plsc.VectorSubcoreMesh(core_axis_name="c", subcore_axis_name="s")


_SC_PARAMS = pltpu.CompilerParams(use_tc_tiling_on_sc=False)


Q = 4


def _sc_gather(table, idx, D, dtype=jnp.float32):
    B = idx.shape[0]
    b_per_w = B // NW
    n_chunks = b_per_w // CHUNK
    n_bodies = n_chunks // Q

    @functools.partial(
        pl.kernel,
        mesh=_mesh(),
        compiler_params=_SC_PARAMS,
        out_type=jax.ShapeDtypeStruct((B, D), dtype),
        scratch_types=[
            pltpu.VMEM((n_chunks, CHUNK), jnp.int32),
            pltpu.VMEM((Q, CHUNK, D), dtype),
            pltpu.SemaphoreType.DMA,
            pltpu.SemaphoreType.DMA,
        ],
    )
    def k(table_hbm, idx_hbm, out_hbm, idx_v, rows_v, gsem, osem):
        wid = lax.axis_index("s") * NC + lax.axis_index("c")
        base = wid * b_per_w
        pltpu.sync_copy(idx_hbm.at[wid], idx_v)

        def body(i, _):
            gs = [pltpu.async_copy(table_hbm.at[idx_v.at[i * Q + q]],
                                   rows_v.at[q], gsem) for q in range(Q)]
            for g in gs:
                g.wait()
            os = [pltpu.async_copy(
                rows_v.at[q],
                out_hbm.at[pl.ds(base + (i * Q + q) * CHUNK, CHUNK)],
                osem) for q in range(Q)]
            for o in os:
                o.wait()
            return 0

        lax.fori_loop(0, n_bodies, body, 0)

    return k(table, idx.reshape(NW, n_chunks, CHUNK))


def _sc_scatter_add(rows, idx, D):
    Dh = D // 2
    per_tile = E_PAD // NS
    n_chunks = per_tile // CHUNK
    rpt = V_PAD // NS

    @functools.partial(
        pl.kernel,
        mesh=_mesh(),
        compiler_params=_SC_PARAMS,
        out_type=jax.ShapeDtypeStruct((V_PAD, D), jnp.float32),
        scratch_types=[
            pltpu.VMEM((n_chunks, CHUNK), jnp.int32),
            pltpu.VMEM((Q, CHUNK, Dh), jnp.float32),
            pltpu.VMEM_SHARED((V_PAD, Dh), jnp.float32),
            pltpu.SemaphoreType.DMA,
            pltpu.SemaphoreType.DMA,
        ],
    )
    def k(rows_hbm, idx_hbm, zeros_hbm, out_hbm, idx_v, rows_v, acc_sh,
          lsem, ssem):
        c = lax.axis_index("c")
        s = lax.axis_index("s")
        col = c * Dh

        pltpu.sync_copy(zeros_hbm.at[pl.ds(s * rpt, rpt)],
                        acc_sh.at[pl.ds(s * rpt, rpt)])
        pltpu.sync_copy(idx_hbm.at[s], idx_v)
        plsc.subcore_barrier()

        base = s * per_tile

        def body(i, _):
            ls = [pltpu.async_copy(
                rows_hbm.at[pl.ds(base + (i * Q + q) * CHUNK, CHUNK),
                            pl.ds(col, Dh)],
                rows_v.at[q], lsem) for q in range(Q)]
            for l in ls:
                l.wait()
            ss = [pltpu.async_copy(rows_v.at[q],
                                   acc_sh.at[idx_v.at[i * Q + q]],
                                   ssem, add=True) for q in range(Q)]
            for x in ss:
                x.wait()
            return 0

        lax.fori_loop(0, n_chunks // Q, body, 0)
        plsc.subcore_barrier()
        pltpu.sync_copy(acc_sh.at[pl.ds(s * rpt, rpt)],
                        out_hbm.at[pl.ds(s * rpt, rpt), pl.ds(col, Dh)])

    zeros = jnp.zeros((V_PAD, Dh), jnp.float32)
    return k(rows, idx.reshape(NS, n_chunks, CHUNK), zeros)


def _edge_mlp_body(g1, g2, g3, er, ea, s, ws, bs, ef_ref, sub_ref):
    g1f = g1[...].astype(jnp.float32)
    g2f = g2[...].astype(jnp.float32)
    h = (jnp.dot(g1f, ws[0][:128], preferred_element_type=jnp.float32)
         + jnp.dot(g2f, ws[0][128:256], preferred_element_type=jnp.float32)
         + jnp.dot(er[...], ws[0][256:288], preferred_element_type=jnp.float32)
         + jnp.dot(ea[...], ws[0][288:320], preferred_element_type=jnp.float32)
         + bs[0][...])
    h = jnp.maximum(h, 0.0)
    for i in range(1, 6):
        h = jnp.dot(h, ws[i], preferred_element_type=jnp.float32) + bs[i][...]
        if i < 5:
            h = jnp.maximum(h, 0.0)
    ef_ref[...] = h
    sub_ref[:, :128] = g3[...].astype(jnp.float32) * s
    sub_ref[:, 128:160] = er[...] * s
    sub_ref[:, 160:192] = ea[...] * s


def _tc_edge(g1, g2, g3, er, ea, invsrc, edge_params):
    EB = 640
    grid = E_PAD // EB
    ws = [p["w"] for p in edge_params]
    bs = [p["b"].reshape(1, -1) for p in edge_params]

    def body(g1r, g2r, g3r, err, ear, invr, w0, w1, w2, w3, w4, w5,
             b0, b1, b2, b3, b4, b5, ef_ref, sub_ref):
        _edge_mlp_body(g1r, g2r, g3r, err, ear, invr[...],
                       [w0[...], w1[...], w2[...], w3[...], w4[...], w5[...]],
                       [b0, b1, b2, b3, b4, b5], ef_ref, sub_ref)

    def full(a):
        return pl.BlockSpec(a.shape, lambda i: (0,) * a.ndim)

    eb = lambda d: pl.BlockSpec((EB, d), lambda i: (i, 0))
    return pl.pallas_call(
        body,
        grid=(grid,),
        in_specs=[eb(128), eb(128), eb(128), eb(32), eb(32), eb(1)]
                 + [full(w) for w in ws] + [full(b) for b in bs],
        out_specs=[eb(64), eb(192)],
        out_shape=[jax.ShapeDtypeStruct((E_PAD, 64), jnp.float32),
                   jax.ShapeDtypeStruct((E_PAD, 192), jnp.float32)],
    )(g1, g2, g3, er, ea, invsrc, *ws, *bs)


def _tc_node(msgs_a, x, deg, node_params):
    NB = 1000
    grid = N // NB
    w = node_params["w"]
    b = node_params["b"].reshape(1, -1)

    def body(ma, xr, dr, wr, br, out_ref):
        d = dr[...]
        isq = lax.rsqrt(d)
        m = ma[...] * isq
        m128 = m[:, :128] + xr[...] / d
        mfull = jnp.concatenate([m128, m[:, 128:]], axis=1)
        z = jnp.dot(mfull, wr[...], preferred_element_type=jnp.float32) + br[...]
        out_ref[...] = z * jax.nn.sigmoid(z)

    nb = lambda d: pl.BlockSpec((NB, d), lambda i: (i, 0))

    def full(a):
        return pl.BlockSpec(a.shape, lambda i: (0,) * a.ndim)

    return pl.pallas_call(
        body,
        grid=(grid,),
        in_specs=[nb(192), nb(128), nb(1), full(w), full(b)],
        out_specs=nb(128),
        out_shape=jax.ShapeDtypeStruct((N, 128), jnp.float32),
    )(msgs_a, x, deg, w, b)


def _tc_heads(x, atom_type, heads):
    NB = 1000
    grid = N // NB
    ws = [p["w"] for h in heads for p in h]
    bs = [p["b"].reshape(1, -1) for h in heads for p in h]

    def body(xr, ar, *rest):
        refs = rest[:-1]
        out_ref = rest[-1]
        outs = []
        for t in range(2):
            h = xr[...]
            for i in range(5):
                h = (jnp.dot(h, refs[5 * t + i][...],
                             preferred_element_type=jnp.float32)
                     + refs[10 + 5 * t + i][...])
                if i < 4:
                    h = jnp.maximum(h, 0.0)
            outs.append(h)
        out_ref[...] = jnp.where(ar[...] == 0, outs[0], outs[1])

    nb = lambda d: pl.BlockSpec((NB, d), lambda i: (i, 0))

    def full(a):
        return pl.BlockSpec(a.shape, lambda i: (0,) * a.ndim)

    return pl.pallas_call(
        body,
        grid=(grid,),
        in_specs=[nb(128), nb(1)] + [full(w) for w in ws] + [full(b) for b in bs],
        out_specs=nb(81),
        out_shape=jax.ShapeDtypeStruct((N, 81), jnp.float32),
    )(x, atom_type, *ws, *bs)


def kernel(node_env, edge_radial, edge_angular, params, edge_index, atom_type):
    x = node_env
    src = edge_index[0].astype(jnp.int32)
    dst = edge_index[1].astype(jnp.int32)
    pad = E_PAD - E
    src_g = jnp.pad(src, (0, pad))
    dst_g = jnp.pad(dst, (0, pad))
    dst_s = jnp.pad(dst, (0, pad), constant_values=N)
    er_p = jnp.pad(edge_radial, ((0, pad), (0, 0)))
    ea_p = jnp.pad(edge_angular, ((0, pad), (0, 0)))

    hist = _sc_scatter_add(jnp.ones((E_PAD, 16), jnp.float32), dst_s, D=16)
    deg = hist[:N, 0]
    invdeg = (1.0 / deg)[:, None]
    deg2d = deg[:, None]

    pretab = jnp.concatenate(
        [src[:N, None], lax.bitcast_convert_type(invdeg, jnp.int32),
         jnp.zeros((N, 14), jnp.int32)], axis=1)
    pre = _sc_gather(pretab, src_g, D=16, dtype=jnp.int32)
    idx2 = pre[:, 0]
    invsrc = lax.bitcast_convert_type(pre[:, 1:2], jnp.float32)

    idx_all = jnp.concatenate([src_g, dst_g, idx2])
    for lp in params["mp"]:
        g = _sc_gather(x.astype(jnp.bfloat16), idx_all, D=128,
                       dtype=jnp.bfloat16)
        g1, g2, g3 = g[:E_PAD], g[E_PAD:2 * E_PAD], g[2 * E_PAD:]
        ef, sub = _tc_edge(g1, g2, g3, er_p, ea_p, invsrc, lp["edge"])
        msgs = _sc_scatter_add(sub, dst_s, D=192)
        x = _tc_node(msgs[:N], x, deg2d, lp["node"])
        er_p, ea_p = ef[:, :32], ef[:, 32:]

    out = _tc_heads(x, atom_type.astype(jnp.int32)[:, None], params["heads"])
    return out.reshape(N, 9, 9)


# speedup vs baseline: 1.0003x; 1.0003x over previous
import functools

import jax
import jax.numpy as jnp
from jax import lax
from jax.experimental import pallas as pl
from jax.experimental.pallas import tpu as pltpu
from jax.experimental.pallas import tpu_sc as plsc

N = 10000
E = 160000
E_PAD = 163840
V_PAD = 10240
NC, NS = 2, 16
NW = NC * NS
CHUNK = 128


def _mesh():
    return plsc.VectorSubcoreMesh(core_axis_name="c", subcore_axis_name="s")


_SC_PARAMS = pltpu.CompilerParams(use_tc_tiling_on_sc=False)


Q = 4


def _sc_gather(table, idx, D, dtype=jnp.float32):
    B = idx.shape[0]
    b_per_w = B // NW
    n_chunks = b_per_w // CHUNK
    n_bodies = n_chunks // Q

    @functools.partial(
        pl.kernel,
        mesh=_mesh(),
        compiler_params=_SC_PARAMS,
        out_type=jax.ShapeDtypeStruct((B, D), dtype),
        scratch_types=[
            pltpu.VMEM((n_chunks, CHUNK), jnp.int32),
            pltpu.VMEM((2, Q, CHUNK, D), dtype),
            pltpu.SemaphoreType.DMA,
            pltpu.SemaphoreType.DMA,
        ],
    )
    def k(table_hbm, idx_hbm, out_hbm, idx_v, rows_v, gsem, osem):
        wid = lax.axis_index("s") * NC + lax.axis_index("c")
        base = wid * b_per_w
        pltpu.sync_copy(idx_hbm.at[wid], idx_v)

        def issue_gathers(grp, bank):
            for q in range(Q):
                pltpu.async_copy(table_hbm.at[idx_v.at[grp * Q + q]],
                                 rows_v.at[bank, q], gsem)

        def drain(sem):
            for _ in range(Q):
                pltpu.make_async_copy(rows_v.at[0, 0],
                                      out_hbm.at[pl.ds(base, CHUNK)],
                                      sem).wait()

        issue_gathers(0, 0)

        def body(i, _):
            bank = lax.rem(i, 2)
            drain(gsem)

            @pl.when(i > 0)
            def _():
                drain(osem)

            @pl.when(i + 1 < n_bodies)
            def _():
                issue_gathers(i + 1, 1 - bank)

            for q in range(Q):
                pltpu.async_copy(
                    rows_v.at[bank, q],
                    out_hbm.at[pl.ds(base + (i * Q + q) * CHUNK, CHUNK)],
                    osem)
            return 0

        lax.fori_loop(0, n_bodies, body, 0)
        drain(osem)

    return k(table, idx.reshape(NW, n_chunks, CHUNK))


def _sc_scatter_add(rows, idx, D):
    Dh = D // 2
    per_tile = E_PAD // NS
    n_chunks = per_tile // CHUNK
    rpt = V_PAD // NS
    QS = 2

    @functools.partial(
        pl.kernel,
        mesh=_mesh(),
        compiler_params=_SC_PARAMS,
        out_type=jax.ShapeDtypeStruct((V_PAD, D), jnp.float32),
        scratch_types=[
            pltpu.VMEM((n_chunks, CHUNK), jnp.int32),
            pltpu.VMEM((2, QS, CHUNK, Dh), jnp.float32),
            pltpu.VMEM_SHARED((V_PAD, Dh), jnp.float32),
            pltpu.SemaphoreType.DMA,
            pltpu.SemaphoreType.DMA,
        ],
    )
    def k(rows_hbm, idx_hbm, zeros_hbm, out_hbm, idx_v, rows_v, acc_sh,
          lsem, ssem):
        c = lax.axis_index("c")
        s = lax.axis_index("s")
        col = c * Dh
        n_bodies = n_chunks // QS

        pltpu.sync_copy(zeros_hbm.at[pl.ds(s * rpt, rpt)],
                        acc_sh.at[pl.ds(s * rpt, rpt)])
        pltpu.sync_copy(idx_hbm.at[s], idx_v)
        plsc.subcore_barrier()

        base = s * per_tile

        def issue_loads(grp, bank):
            for q in range(QS):
                pltpu.async_copy(
                    rows_hbm.at[pl.ds(base + (grp * QS + q) * CHUNK, CHUNK),
                                pl.ds(col, Dh)],
                    rows_v.at[bank, q], lsem)

        def drain(sem):
            for _ in range(QS):
                pltpu.make_async_copy(rows_v.at[0, 0],
                                      acc_sh.at[pl.ds(0, CHUNK)], sem).wait()

        issue_loads(0, 0)

        def body(i, _):
            bank = lax.rem(i, 2)
            drain(lsem)

            @pl.when(i > 0)
            def _():
                drain(ssem)

            @pl.when(i + 1 < n_bodies)
            def _():
                issue_loads(i + 1, 1 - bank)

            for q in range(QS):
                pltpu.async_copy(rows_v.at[bank, q],
                                 acc_sh.at[idx_v.at[i * QS + q]],
                                 ssem, add=True)
            return 0

        lax.fori_loop(0, n_bodies, body, 0)
        drain(ssem)
        plsc.subcore_barrier()
        pltpu.sync_copy(acc_sh.at[pl.ds(s * rpt, rpt)],
                        out_hbm.at[pl.ds(s * rpt, rpt), pl.ds(col, Dh)])

    zeros = jnp.zeros((V_PAD, Dh), jnp.float32)
    return k(rows, idx.reshape(NS, n_chunks, CHUNK), zeros)


def _edge_mlp_body(g1, g2, g3, er, ea, s, ws, bs, ef_ref, sub_ref):
    g1f = g1[...].astype(jnp.float32)
    g2f = g2[...].astype(jnp.float32)
    h = (jnp.dot(g1f, ws[0][:128], preferred_element_type=jnp.float32)
         + jnp.dot(g2f, ws[0][128:256], preferred_element_type=jnp.float32)
         + jnp.dot(er[...], ws[0][256:288], preferred_element_type=jnp.float32)
         + jnp.dot(ea[...], ws[0][288:320], preferred_element_type=jnp.float32)
         + bs[0][...])
    h = jnp.maximum(h, 0.0)
    for i in range(1, 6):
        h = jnp.dot(h, ws[i], preferred_element_type=jnp.float32) + bs[i][...]
        if i < 5:
            h = jnp.maximum(h, 0.0)
    ef_ref[...] = h
    sub_ref[:, :128] = g3[...].astype(jnp.float32) * s
    sub_ref[:, 128:160] = er[...] * s
    sub_ref[:, 160:192] = ea[...] * s


def _tc_edge(g1, g2, g3, er, ea, invsrc, edge_params):
    EB = 640
    grid = E_PAD // EB
    ws = [p["w"] for p in edge_params]
    bs = [p["b"].reshape(1, -1) for p in edge_params]

    def body(g1r, g2r, g3r, err, ear, invr, w0, w1, w2, w3, w4, w5,
             b0, b1, b2, b3, b4, b5, ef_ref, sub_ref):
        _edge_mlp_body(g1r, g2r, g3r, err, ear, invr[...],
                       [w0[...], w1[...], w2[...], w3[...], w4[...], w5[...]],
                       [b0, b1, b2, b3, b4, b5], ef_ref, sub_ref)

    def full(a):
        return pl.BlockSpec(a.shape, lambda i: (0,) * a.ndim)

    eb = lambda d: pl.BlockSpec((EB, d), lambda i: (i, 0))
    return pl.pallas_call(
        body,
        grid=(grid,),
        in_specs=[eb(128), eb(128), eb(128), eb(32), eb(32), eb(1)]
                 + [full(w) for w in ws] + [full(b) for b in bs],
        out_specs=[eb(64), eb(192)],
        out_shape=[jax.ShapeDtypeStruct((E_PAD, 64), jnp.float32),
                   jax.ShapeDtypeStruct((E_PAD, 192), jnp.float32)],
    )(g1, g2, g3, er, ea, invsrc, *ws, *bs)


def _tc_node(msgs_a, x, deg, node_params):
    NB = 1000
    grid = N // NB
    w = node_params["w"]
    b = node_params["b"].reshape(1, -1)

    def body(ma, xr, dr, wr, br, out_ref):
        d = dr[...]
        isq = lax.rsqrt(d)
        m = ma[...] * isq
        m128 = m[:, :128] + xr[...] / d
        mfull = jnp.concatenate([m128, m[:, 128:]], axis=1)
        z = jnp.dot(mfull, wr[...], preferred_element_type=jnp.float32) + br[...]
        out_ref[...] = z * jax.nn.sigmoid(z)

    nb = lambda d: pl.BlockSpec((NB, d), lambda i: (i, 0))

    def full(a):
        return pl.BlockSpec(a.shape, lambda i: (0,) * a.ndim)

    return pl.pallas_call(
        body,
        grid=(grid,),
        in_specs=[nb(192), nb(128), nb(1), full(w), full(b)],
        out_specs=nb(128),
        out_shape=jax.ShapeDtypeStruct((N, 128), jnp.float32),
    )(msgs_a, x, deg, w, b)


def _tc_heads(x, atom_type, heads):
    NB = 1000
    grid = N // NB
    ws = [p["w"] for h in heads for p in h]
    bs = [p["b"].reshape(1, -1) for h in heads for p in h]

    def body(xr, ar, *rest):
        refs = rest[:-1]
        out_ref = rest[-1]
        outs = []
        for t in range(2):
            h = xr[...]
            for i in range(5):
                h = (jnp.dot(h, refs[5 * t + i][...],
                             preferred_element_type=jnp.float32)
                     + refs[10 + 5 * t + i][...])
                if i < 4:
                    h = jnp.maximum(h, 0.0)
            outs.append(h)
        out_ref[...] = jnp.where(ar[...] == 0, outs[0], outs[1])

    nb = lambda d: pl.BlockSpec((NB, d), lambda i: (i, 0))

    def full(a):
        return pl.BlockSpec(a.shape, lambda i: (0,) * a.ndim)

    return pl.pallas_call(
        body,
        grid=(grid,),
        in_specs=[nb(128), nb(1)] + [full(w) for w in ws] + [full(b) for b in bs],
        out_specs=nb(81),
        out_shape=jax.ShapeDtypeStruct((N, 81), jnp.float32),
    )(x, atom_type, *ws, *bs)


def kernel(node_env, edge_radial, edge_angular, params, edge_index, atom_type):
    x = node_env
    src = edge_index[0].astype(jnp.int32)
    dst = edge_index[1].astype(jnp.int32)
    pad = E_PAD - E
    src_g = jnp.pad(src, (0, pad))
    dst_g = jnp.pad(dst, (0, pad))
    dst_s = jnp.pad(dst, (0, pad), constant_values=N)
    er_p = jnp.pad(edge_radial, ((0, pad), (0, 0)))
    ea_p = jnp.pad(edge_angular, ((0, pad), (0, 0)))

    hist = _sc_scatter_add(jnp.ones((E_PAD, 16), jnp.float32), dst_s, D=16)
    deg = hist[:N, 0]
    invdeg = (1.0 / deg)[:, None]
    deg2d = deg[:, None]

    pretab = jnp.concatenate(
        [src[:N, None], lax.bitcast_convert_type(invdeg, jnp.int32),
         jnp.zeros((N, 14), jnp.int32)], axis=1)
    pre = _sc_gather(pretab, src_g, D=16, dtype=jnp.int32)
    idx2 = pre[:, 0]
    invsrc = lax.bitcast_convert_type(pre[:, 1:2], jnp.float32)

    idx_all = jnp.concatenate([src_g, dst_g, idx2])
    for lp in params["mp"]:
        g = _sc_gather(x.astype(jnp.bfloat16), idx_all, D=128,
                       dtype=jnp.bfloat16)
        g1, g2, g3 = g[:E_PAD], g[E_PAD:2 * E_PAD], g[2 * E_PAD:]
        ef, sub = _tc_edge(g1, g2, g3, er_p, ea_p, invsrc, lp["edge"])
        msgs = _sc_scatter_add(sub, dst_s, D=192)
        x = _tc_node(msgs[:N], x, deg2d, lp["node"])
        er_p, ea_p = ef[:, :32], ef[:, 32:]

    out = _tc_heads(x, atom_type.astype(jnp.int32)[:, None], params["heads"])
    return out.reshape(N, 9, 9)


# speedup vs baseline: 1.1658x; 1.1654x over previous
import functools

import jax
import jax.numpy as jnp
from jax import lax
from jax.experimental import pallas as pl
from jax.experimental.pallas import tpu as pltpu
from jax.experimental.pallas import tpu_sc as plsc

N = 10000
E = 160000
E_PAD = 163840
V_PAD = 10240
NC, NS = 2, 16
NW = NC * NS
CHUNK = 128


def _mesh():
    return plsc.VectorSubcoreMesh(core_axis_name="c", subcore_axis_name="s")


_SC_PARAMS = pltpu.CompilerParams(use_tc_tiling_on_sc=False)


Q = 4


def _sc_gather(table, idx, D, dtype=jnp.float32):
    B = idx.shape[0]
    V = table.shape[0]
    vpt = V // NS
    b_per_w = B // NW
    n_chunks = b_per_w // CHUNK
    n_bodies = n_chunks // Q

    @functools.partial(
        pl.kernel,
        mesh=_mesh(),
        compiler_params=_SC_PARAMS,
        out_type=jax.ShapeDtypeStruct((B, D), dtype),
        scratch_types=[
            pltpu.VMEM((n_chunks, CHUNK), jnp.int32),
            pltpu.VMEM((Q, CHUNK, D), dtype),
            pltpu.VMEM_SHARED((V, D), dtype),
            pltpu.SemaphoreType.DMA,
            pltpu.SemaphoreType.DMA,
        ],
    )
    def k(table_hbm, idx_hbm, out_hbm, idx_v, rows_v, tab_sh, gsem, osem):
        c = lax.axis_index("c")
        s = lax.axis_index("s")
        wid = s * NC + c
        base = wid * b_per_w
        pltpu.sync_copy(table_hbm.at[pl.ds(s * vpt, vpt)],
                        tab_sh.at[pl.ds(s * vpt, vpt)])
        pltpu.sync_copy(idx_hbm.at[wid], idx_v)
        plsc.subcore_barrier()

        def body(i, _):
            gs = [pltpu.async_copy(tab_sh.at[idx_v.at[i * Q + q]],
                                   rows_v.at[q], gsem) for q in range(Q)]
            for g in gs:
                g.wait()
            os = [pltpu.async_copy(
                rows_v.at[q],
                out_hbm.at[pl.ds(base + (i * Q + q) * CHUNK, CHUNK)],
                osem) for q in range(Q)]
            for o in os:
                o.wait()
            return 0

        lax.fori_loop(0, n_bodies, body, 0)

    return k(table, idx.reshape(NW, n_chunks, CHUNK))


def _sc_scatter_add(rows, idx, D):
    Dh = D // 2
    per_tile = E_PAD // NS
    n_chunks = per_tile // CHUNK
    rpt = V_PAD // NS
    QS = 2

    @functools.partial(
        pl.kernel,
        mesh=_mesh(),
        compiler_params=_SC_PARAMS,
        out_type=jax.ShapeDtypeStruct((V_PAD, D), jnp.float32),
        scratch_types=[
            pltpu.VMEM((n_chunks, CHUNK), jnp.int32),
            pltpu.VMEM((2, QS, CHUNK, Dh), jnp.float32),
            pltpu.VMEM_SHARED((V_PAD, Dh), jnp.float32),
            pltpu.SemaphoreType.DMA,
            pltpu.SemaphoreType.DMA,
        ],
    )
    def k(rows_hbm, idx_hbm, zeros_hbm, out_hbm, idx_v, rows_v, acc_sh,
          lsem, ssem):
        c = lax.axis_index("c")
        s = lax.axis_index("s")
        col = c * Dh
        n_bodies = n_chunks // QS

        pltpu.sync_copy(zeros_hbm.at[pl.ds(s * rpt, rpt)],
                        acc_sh.at[pl.ds(s * rpt, rpt)])
        pltpu.sync_copy(idx_hbm.at[s], idx_v)
        plsc.subcore_barrier()

        base = s * per_tile

        def issue_loads(grp, bank):
            for q in range(QS):
                pltpu.async_copy(
                    rows_hbm.at[pl.ds(base + (grp * QS + q) * CHUNK, CHUNK),
                                pl.ds(col, Dh)],
                    rows_v.at[bank, q], lsem)

        def drain(sem):
            for _ in range(QS):
                pltpu.make_async_copy(rows_v.at[0, 0],
                                      acc_sh.at[pl.ds(0, CHUNK)], sem).wait()

        issue_loads(0, 0)

        def body(i, _):
            bank = lax.rem(i, 2)
            drain(lsem)

            @pl.when(i > 0)
            def _():
                drain(ssem)

            @pl.when(i + 1 < n_bodies)
            def _():
                issue_loads(i + 1, 1 - bank)

            for q in range(QS):
                pltpu.async_copy(rows_v.at[bank, q],
                                 acc_sh.at[idx_v.at[i * QS + q]],
                                 ssem, add=True)
            return 0

        lax.fori_loop(0, n_bodies, body, 0)
        drain(ssem)
        plsc.subcore_barrier()
        pltpu.sync_copy(acc_sh.at[pl.ds(s * rpt, rpt)],
                        out_hbm.at[pl.ds(s * rpt, rpt), pl.ds(col, Dh)])

    zeros = jnp.zeros((V_PAD, Dh), jnp.float32)
    return k(rows, idx.reshape(NS, n_chunks, CHUNK), zeros)


def _edge_mlp_body(g1, g2, g3, er, ea, s, ws, bs, ef_ref, sub_ref):
    g1f = g1[...].astype(jnp.float32)
    g2f = g2[...].astype(jnp.float32)
    h = (jnp.dot(g1f, ws[0][:128], preferred_element_type=jnp.float32)
         + jnp.dot(g2f, ws[0][128:256], preferred_element_type=jnp.float32)
         + jnp.dot(er[...], ws[0][256:288], preferred_element_type=jnp.float32)
         + jnp.dot(ea[...], ws[0][288:320], preferred_element_type=jnp.float32)
         + bs[0][...])
    h = jnp.maximum(h, 0.0)
    for i in range(1, 6):
        h = jnp.dot(h, ws[i], preferred_element_type=jnp.float32) + bs[i][...]
        if i < 5:
            h = jnp.maximum(h, 0.0)
    ef_ref[...] = h
    sub_ref[:, :128] = g3[...].astype(jnp.float32) * s
    sub_ref[:, 128:160] = er[...] * s
    sub_ref[:, 160:192] = ea[...] * s


def _tc_edge(g1, g2, g3, er, ea, invsrc, edge_params):
    EB = 640
    grid = E_PAD // EB
    ws = [p["w"] for p in edge_params]
    bs = [p["b"].reshape(1, -1) for p in edge_params]

    def body(g1r, g2r, g3r, err, ear, invr, w0, w1, w2, w3, w4, w5,
             b0, b1, b2, b3, b4, b5, ef_ref, sub_ref):
        _edge_mlp_body(g1r, g2r, g3r, err, ear, invr[...],
                       [w0[...], w1[...], w2[...], w3[...], w4[...], w5[...]],
                       [b0, b1, b2, b3, b4, b5], ef_ref, sub_ref)

    def full(a):
        return pl.BlockSpec(a.shape, lambda i: (0,) * a.ndim)

    eb = lambda d: pl.BlockSpec((EB, d), lambda i: (i, 0))
    return pl.pallas_call(
        body,
        grid=(grid,),
        in_specs=[eb(128), eb(128), eb(128), eb(32), eb(32), eb(1)]
                 + [full(w) for w in ws] + [full(b) for b in bs],
        out_specs=[eb(64), eb(192)],
        out_shape=[jax.ShapeDtypeStruct((E_PAD, 64), jnp.float32),
                   jax.ShapeDtypeStruct((E_PAD, 192), jnp.float32)],
    )(g1, g2, g3, er, ea, invsrc, *ws, *bs)


def _tc_node(msgs_a, x, deg, node_params):
    NB = 1000
    grid = N // NB
    w = node_params["w"]
    b = node_params["b"].reshape(1, -1)

    def body(ma, xr, dr, wr, br, out_ref):
        d = dr[...]
        isq = lax.rsqrt(d)
        m = ma[...] * isq
        m128 = m[:, :128] + xr[...] / d
        mfull = jnp.concatenate([m128, m[:, 128:]], axis=1)
        z = jnp.dot(mfull, wr[...], preferred_element_type=jnp.float32) + br[...]
        out_ref[...] = z * jax.nn.sigmoid(z)

    nb = lambda d: pl.BlockSpec((NB, d), lambda i: (i, 0))

    def full(a):
        return pl.BlockSpec(a.shape, lambda i: (0,) * a.ndim)

    return pl.pallas_call(
        body,
        grid=(grid,),
        in_specs=[nb(192), nb(128), nb(1), full(w), full(b)],
        out_specs=nb(128),
        out_shape=jax.ShapeDtypeStruct((N, 128), jnp.float32),
    )(msgs_a, x, deg, w, b)


def _tc_heads(x, atom_type, heads):
    NB = 1000
    grid = N // NB
    ws = [p["w"] for h in heads for p in h]
    bs = [p["b"].reshape(1, -1) for h in heads for p in h]

    def body(xr, ar, *rest):
        refs = rest[:-1]
        out_ref = rest[-1]
        outs = []
        for t in range(2):
            h = xr[...]
            for i in range(5):
                h = (jnp.dot(h, refs[5 * t + i][...],
                             preferred_element_type=jnp.float32)
                     + refs[10 + 5 * t + i][...])
                if i < 4:
                    h = jnp.maximum(h, 0.0)
            outs.append(h)
        out_ref[...] = jnp.where(ar[...] == 0, outs[0], outs[1])

    nb = lambda d: pl.BlockSpec((NB, d), lambda i: (i, 0))

    def full(a):
        return pl.BlockSpec(a.shape, lambda i: (0,) * a.ndim)

    return pl.pallas_call(
        body,
        grid=(grid,),
        in_specs=[nb(128), nb(1)] + [full(w) for w in ws] + [full(b) for b in bs],
        out_specs=nb(81),
        out_shape=jax.ShapeDtypeStruct((N, 81), jnp.float32),
    )(x, atom_type, *ws, *bs)


def kernel(node_env, edge_radial, edge_angular, params, edge_index, atom_type):
    x = node_env
    src = edge_index[0].astype(jnp.int32)
    dst = edge_index[1].astype(jnp.int32)
    pad = E_PAD - E
    src_g = jnp.pad(src, (0, pad))
    dst_g = jnp.pad(dst, (0, pad))
    dst_s = jnp.pad(dst, (0, pad), constant_values=N)
    er_p = jnp.pad(edge_radial, ((0, pad), (0, 0)))
    ea_p = jnp.pad(edge_angular, ((0, pad), (0, 0)))

    hist = _sc_scatter_add(jnp.ones((E_PAD, 16), jnp.float32), dst_s, D=16)
    deg = hist[:N, 0]
    invdeg = (1.0 / deg)[:, None]
    deg2d = deg[:, None]

    pretab = jnp.concatenate(
        [src[:N, None], lax.bitcast_convert_type(invdeg, jnp.int32),
         jnp.zeros((N, 14), jnp.int32)], axis=1)
    pre = _sc_gather(pretab, src_g, D=16, dtype=jnp.int32)
    idx2 = pre[:, 0]
    invsrc = lax.bitcast_convert_type(pre[:, 1:2], jnp.float32)

    idx_all = jnp.concatenate([src_g, dst_g, idx2])
    for lp in params["mp"]:
        g = _sc_gather(x.astype(jnp.bfloat16), idx_all, D=128,
                       dtype=jnp.bfloat16)
        g1, g2, g3 = g[:E_PAD], g[E_PAD:2 * E_PAD], g[2 * E_PAD:]
        ef, sub = _tc_edge(g1, g2, g3, er_p, ea_p, invsrc, lp["edge"])
        msgs = _sc_scatter_add(sub, dst_s, D=192)
        x = _tc_node(msgs[:N], x, deg2d, lp["node"])
        er_p, ea_p = ef[:, :32], ef[:, 32:]

    out = _tc_heads(x, atom_type.astype(jnp.int32)[:, None], params["heads"])
    return out.reshape(N, 9, 9)


# speedup vs baseline: 1.1668x; 1.0009x over previous
import functools

import jax
import jax.numpy as jnp
from jax import lax
from jax.experimental import pallas as pl
from jax.experimental.pallas import tpu as pltpu
from jax.experimental.pallas import tpu_sc as plsc

N = 10000
E = 160000
E_PAD = 163840
V_PAD = 10240
NC, NS = 2, 16
NW = NC * NS
CHUNK = 128


def _mesh():
    return plsc.VectorSubcoreMesh(core_axis_name="c", subcore_axis_name="s")


_SC_PARAMS = pltpu.CompilerParams(use_tc_tiling_on_sc=False)


Q = 8


def _sc_gather(table, idx, D, dtype=jnp.float32):
    B = idx.shape[0]
    V = table.shape[0]
    vpt = V // NS
    b_per_w = B // NW
    n_chunks = b_per_w // CHUNK
    n_bodies = n_chunks // Q

    @functools.partial(
        pl.kernel,
        mesh=_mesh(),
        compiler_params=_SC_PARAMS,
        out_type=jax.ShapeDtypeStruct((B, D), dtype),
        scratch_types=[
            pltpu.VMEM((n_chunks, CHUNK), jnp.int32),
            pltpu.VMEM((Q, CHUNK, D), dtype),
            pltpu.VMEM_SHARED((V, D), dtype),
            pltpu.SemaphoreType.DMA,
            pltpu.SemaphoreType.DMA,
        ],
    )
    def k(table_hbm, idx_hbm, out_hbm, idx_v, rows_v, tab_sh, gsem, osem):
        c = lax.axis_index("c")
        s = lax.axis_index("s")
        wid = s * NC + c
        base = wid * b_per_w
        pltpu.sync_copy(table_hbm.at[pl.ds(s * vpt, vpt)],
                        tab_sh.at[pl.ds(s * vpt, vpt)])
        pltpu.sync_copy(idx_hbm.at[wid], idx_v)
        plsc.subcore_barrier()

        def body(i, _):
            gs = [pltpu.async_copy(tab_sh.at[idx_v.at[i * Q + q]],
                                   rows_v.at[q], gsem) for q in range(Q)]
            for g in gs:
                g.wait()
            os = [pltpu.async_copy(
                rows_v.at[q],
                out_hbm.at[pl.ds(base + (i * Q + q) * CHUNK, CHUNK)],
                osem) for q in range(Q)]
            for o in os:
                o.wait()
            return 0

        lax.fori_loop(0, n_bodies, body, 0)

    return k(table, idx.reshape(NW, n_chunks, CHUNK))


def _sc_scatter_add(rows, idx, D):
    Dh = D // 2
    per_tile = E_PAD // NS
    n_chunks = per_tile // CHUNK
    rpt = V_PAD // NS
    QS = 2

    @functools.partial(
        pl.kernel,
        mesh=_mesh(),
        compiler_params=_SC_PARAMS,
        out_type=jax.ShapeDtypeStruct((V_PAD, D), jnp.float32),
        scratch_types=[
            pltpu.VMEM((n_chunks, CHUNK), jnp.int32),
            pltpu.VMEM((2, QS, CHUNK, Dh), jnp.float32),
            pltpu.VMEM_SHARED((V_PAD, Dh), jnp.float32),
            pltpu.SemaphoreType.DMA,
            pltpu.SemaphoreType.DMA,
        ],
    )
    def k(rows_hbm, idx_hbm, zeros_hbm, out_hbm, idx_v, rows_v, acc_sh,
          lsem, ssem):
        c = lax.axis_index("c")
        s = lax.axis_index("s")
        col = c * Dh
        n_bodies = n_chunks // QS

        pltpu.sync_copy(zeros_hbm.at[pl.ds(s * rpt, rpt)],
                        acc_sh.at[pl.ds(s * rpt, rpt)])
        pltpu.sync_copy(idx_hbm.at[s], idx_v)
        plsc.subcore_barrier()

        base = s * per_tile

        def issue_loads(grp, bank):
            for q in range(QS):
                pltpu.async_copy(
                    rows_hbm.at[pl.ds(base + (grp * QS + q) * CHUNK, CHUNK),
                                pl.ds(col, Dh)],
                    rows_v.at[bank, q], lsem)

        def drain(sem):
            for _ in range(QS):
                pltpu.make_async_copy(rows_v.at[0, 0],
                                      acc_sh.at[pl.ds(0, CHUNK)], sem).wait()

        issue_loads(0, 0)

        def body(i, _):
            bank = lax.rem(i, 2)
            drain(lsem)

            @pl.when(i > 0)
            def _():
                drain(ssem)

            @pl.when(i + 1 < n_bodies)
            def _():
                issue_loads(i + 1, 1 - bank)

            for q in range(QS):
                pltpu.async_copy(rows_v.at[bank, q],
                                 acc_sh.at[idx_v.at[i * QS + q]],
                                 ssem, add=True)
            return 0

        lax.fori_loop(0, n_bodies, body, 0)
        drain(ssem)
        plsc.subcore_barrier()
        pltpu.sync_copy(acc_sh.at[pl.ds(s * rpt, rpt)],
                        out_hbm.at[pl.ds(s * rpt, rpt), pl.ds(col, Dh)])

    zeros = jnp.zeros((V_PAD, Dh), jnp.float32)
    return k(rows, idx.reshape(NS, n_chunks, CHUNK), zeros)


def _edge_mlp_body(g1, g2, g3, er, ea, s, ws, bs, ef_ref, sub_ref):
    g1f = g1[...].astype(jnp.float32)
    g2f = g2[...].astype(jnp.float32)
    h = (jnp.dot(g1f, ws[0][:128], preferred_element_type=jnp.float32)
         + jnp.dot(g2f, ws[0][128:256], preferred_element_type=jnp.float32)
         + jnp.dot(er[...], ws[0][256:288], preferred_element_type=jnp.float32)
         + jnp.dot(ea[...], ws[0][288:320], preferred_element_type=jnp.float32)
         + bs[0][...])
    h = jnp.maximum(h, 0.0)
    for i in range(1, 6):
        h = jnp.dot(h, ws[i], preferred_element_type=jnp.float32) + bs[i][...]
        if i < 5:
            h = jnp.maximum(h, 0.0)
    ef_ref[...] = h
    sub_ref[:, :128] = g3[...].astype(jnp.float32) * s
    sub_ref[:, 128:160] = er[...] * s
    sub_ref[:, 160:192] = ea[...] * s


def _tc_edge(g1, g2, g3, er, ea, invsrc, edge_params):
    EB = 640
    grid = E_PAD // EB
    ws = [p["w"] for p in edge_params]
    bs = [p["b"].reshape(1, -1) for p in edge_params]

    def body(g1r, g2r, g3r, err, ear, invr, w0, w1, w2, w3, w4, w5,
             b0, b1, b2, b3, b4, b5, ef_ref, sub_ref):
        _edge_mlp_body(g1r, g2r, g3r, err, ear, invr[...],
                       [w0[...], w1[...], w2[...], w3[...], w4[...], w5[...]],
                       [b0, b1, b2, b3, b4, b5], ef_ref, sub_ref)

    def full(a):
        return pl.BlockSpec(a.shape, lambda i: (0,) * a.ndim)

    eb = lambda d: pl.BlockSpec((EB, d), lambda i: (i, 0))
    return pl.pallas_call(
        body,
        grid=(grid,),
        in_specs=[eb(128), eb(128), eb(128), eb(32), eb(32), eb(1)]
                 + [full(w) for w in ws] + [full(b) for b in bs],
        out_specs=[eb(64), eb(192)],
        out_shape=[jax.ShapeDtypeStruct((E_PAD, 64), jnp.float32),
                   jax.ShapeDtypeStruct((E_PAD, 192), jnp.float32)],
    )(g1, g2, g3, er, ea, invsrc, *ws, *bs)


def _tc_node(msgs_a, x, deg, node_params):
    NB = 1000
    grid = N // NB
    w = node_params["w"]
    b = node_params["b"].reshape(1, -1)

    def body(ma, xr, dr, wr, br, out_ref):
        d = dr[...]
        isq = lax.rsqrt(d)
        m = ma[...] * isq
        m128 = m[:, :128] + xr[...] / d
        mfull = jnp.concatenate([m128, m[:, 128:]], axis=1)
        z = jnp.dot(mfull, wr[...], preferred_element_type=jnp.float32) + br[...]
        out_ref[...] = z * jax.nn.sigmoid(z)

    nb = lambda d: pl.BlockSpec((NB, d), lambda i: (i, 0))

    def full(a):
        return pl.BlockSpec(a.shape, lambda i: (0,) * a.ndim)

    return pl.pallas_call(
        body,
        grid=(grid,),
        in_specs=[nb(192), nb(128), nb(1), full(w), full(b)],
        out_specs=nb(128),
        out_shape=jax.ShapeDtypeStruct((N, 128), jnp.float32),
    )(msgs_a, x, deg, w, b)


def _tc_heads(x, atom_type, heads):
    NB = 1000
    grid = N // NB
    ws = [p["w"] for h in heads for p in h]
    bs = [p["b"].reshape(1, -1) for h in heads for p in h]

    def body(xr, ar, *rest):
        refs = rest[:-1]
        out_ref = rest[-1]
        outs = []
        for t in range(2):
            h = xr[...]
            for i in range(5):
                h = (jnp.dot(h, refs[5 * t + i][...],
                             preferred_element_type=jnp.float32)
                     + refs[10 + 5 * t + i][...])
                if i < 4:
                    h = jnp.maximum(h, 0.0)
            outs.append(h)
        out_ref[...] = jnp.where(ar[...] == 0, outs[0], outs[1])

    nb = lambda d: pl.BlockSpec((NB, d), lambda i: (i, 0))

    def full(a):
        return pl.BlockSpec(a.shape, lambda i: (0,) * a.ndim)

    return pl.pallas_call(
        body,
        grid=(grid,),
        in_specs=[nb(128), nb(1)] + [full(w) for w in ws] + [full(b) for b in bs],
        out_specs=nb(81),
        out_shape=jax.ShapeDtypeStruct((N, 81), jnp.float32),
    )(x, atom_type, *ws, *bs)


def kernel(node_env, edge_radial, edge_angular, params, edge_index, atom_type):
    x = node_env
    src = edge_index[0].astype(jnp.int32)
    dst = edge_index[1].astype(jnp.int32)
    pad = E_PAD - E
    src_g = jnp.pad(src, (0, pad))
    dst_g = jnp.pad(dst, (0, pad))
    dst_s = jnp.pad(dst, (0, pad), constant_values=N)
    er_p = jnp.pad(edge_radial, ((0, pad), (0, 0)))
    ea_p = jnp.pad(edge_angular, ((0, pad), (0, 0)))

    hist = _sc_scatter_add(jnp.ones((E_PAD, 16), jnp.float32), dst_s, D=16)
    deg = hist[:N, 0]
    invdeg = (1.0 / deg)[:, None]
    deg2d = deg[:, None]

    pretab = jnp.concatenate(
        [src[:N, None], lax.bitcast_convert_type(invdeg, jnp.int32),
         jnp.zeros((N, 14), jnp.int32)], axis=1)
    pre = _sc_gather(pretab, src_g, D=16, dtype=jnp.int32)
    idx2 = pre[:, 0]
    invsrc = lax.bitcast_convert_type(pre[:, 1:2], jnp.float32)

    idx_all = jnp.concatenate([src_g, dst_g, idx2])
    for lp in params["mp"]:
        g = _sc_gather(x.astype(jnp.bfloat16), idx_all, D=128,
                       dtype=jnp.bfloat16)
        g1, g2, g3 = g[:E_PAD], g[E_PAD:2 * E_PAD], g[2 * E_PAD:]
        ef, sub = _tc_edge(g1, g2, g3, er_p, ea_p, invsrc, lp["edge"])
        msgs = _sc_scatter_add(sub, dst_s, D=192)
        x = _tc_node(msgs[:N], x, deg2d, lp["node"])
        er_p, ea_p = ef[:, :32], ef[:, 32:]

    out = _tc_heads(x, atom_type.astype(jnp.int32)[:, None], params["heads"])
    return out.reshape(N, 9, 9)
